# baseline (speedup 1.0000x reference)
import functools

import jax
import jax.numpy as jnp
from jax import lax
from jax.experimental import pallas as pl
from jax.experimental.pallas import tpu as pltpu
from jax.experimental.pallas import tpu_sc as plsc

NC, NS, LANES = 2, 16, 16
NW = NC * NS
DIM = 4
NPAIR = DIM // 2
WAVE = 10


@functools.lru_cache(maxsize=None)
def _build(B, L, V):
    n_slabs = NW // NPAIR
    slab_b = B // n_slabs
    n_waves = L // WAVE

    mesh = plsc.VectorSubcoreMesh(core_axis_name="c", subcore_axis_name="s")

    @functools.partial(
        pl.kernel,
        out_type=jax.ShapeDtypeStruct((DIM * B,), jnp.float32),
        mesh=mesh,
        scratch_types=[
            pltpu.VMEM((V,), jnp.int32),
            pltpu.VMEM((2, WAVE, slab_b), jnp.int32),
            pltpu.VMEM((2, slab_b), jnp.float32),
            pltpu.SemaphoreType.DMA,
            pltpu.SemaphoreType.DMA,
        ],
        compiler_params=pltpu.CompilerParams(
            needs_layout_passes=False, use_tc_tiling_on_sc=False),
    )
    def kern(tags_hbm, table_hbm, out_hbm, tbl_v, tags_v, acc_v, semt, sem):
        wid = lax.axis_index("s") * NC + lax.axis_index("c")
        p = wid % NPAIR
        b0 = (wid // NPAIR) * slab_b

        tbl_dma = pltpu.async_copy(
            table_hbm.at[pl.ds(p * V, V)], tbl_v, semt)

        def start_wave(w):
            buf = tags_v.at[w % 2]
            return [
                pltpu.async_copy(
                    tags_hbm.at[pl.ds((w * WAVE + i) * B + b0, slab_b)],
                    buf.at[i], sem)
                for i in range(WAVE)
            ]

        hi_mask = jnp.full((16,), -65536, jnp.int32)

        def compute_wave(w):
            buf = w % 2

            @plsc.parallel_loop(0, slab_b, LANES, unroll=8)
            def g_body(base):
                sl = pl.ds(base, 16)
                acc0 = jnp.zeros((16,), jnp.float32)
                acc1 = jnp.zeros((16,), jnp.float32)
                for i in range(WAVE):
                    word = plsc.load_gather(tbl_v, [tags_v[buf, i, sl]])
                    acc0 = acc0 + plsc.bitcast(word << 16, jnp.float32)
                    acc1 = acc1 + plsc.bitcast(word & hi_mask, jnp.float32)
                if w > 0:
                    acc0 = acc0 + acc_v[0, sl]
                    acc1 = acc1 + acc_v[1, sl]
                acc_v[0, sl] = acc0
                acc_v[1, sl] = acc1

        pending = start_wave(0)
        tbl_dma.wait()
        for w in range(n_waves):
            nxt = start_wave(w + 1) if w + 1 < n_waves else None
            for h in pending:
                h.wait()
            compute_wave(w)
            pending = nxt

        pltpu.sync_copy(
            acc_v.at[0], out_hbm.at[pl.ds((2 * p) * B + b0, slab_b)])
        pltpu.sync_copy(
            acc_v.at[1], out_hbm.at[pl.ds((2 * p + 1) * B + b0, slab_b)])

    return kern


def kernel(itemtags, table):
    B, L = itemtags.shape
    V, _ = table.shape
    tags_f = itemtags.T.reshape(L * B)
    packed = jax.lax.bitcast_convert_type(
        table.astype(jnp.bfloat16).reshape(V, NPAIR, 2),
        jnp.int32)
    table_f = packed.T.reshape(NPAIR * V)
    out_f = _build(B, L, V)(tags_f, table_f)
    return out_f.reshape(DIM, B).T
